# baseline (speedup 1.0000x reference)
import functools

import jax
import jax.numpy as jnp
from jax import lax
from jax.experimental import pallas as pl
from jax.experimental.pallas import tpu as pltpu
from jax.experimental.pallas import tpu_sc as plsc

VOC = 100000
DIM = 64
SEQ = 50
BN = 4096
NC = 2
NS = 16
NW = NC * NS
D_PER_W = DIM // NW

_mesh = plsc.VectorSubcoreMesh(core_axis_name="c", subcore_axis_name="s")


@functools.partial(
    pl.kernel,
    mesh=_mesh,
    out_type=jax.ShapeDtypeStruct((SEQ, DIM, BN), jnp.float32),
    compiler_params=pltpu.CompilerParams(needs_layout_passes=False),
    scratch_types=[
        pltpu.VMEM((VOC,), jnp.float32),
        [pltpu.VMEM((BN,), jnp.int32) for _ in range(2)],
        [pltpu.VMEM((BN,), jnp.float32) for _ in range(2)],
        pltpu.VMEM_SHARED((SEQ * BN,), jnp.int32),
        pltpu.SemaphoreType.DMA,
        [pltpu.SemaphoreType.DMA for _ in range(2)],
        [pltpu.SemaphoreType.DMA for _ in range(2)],
    ],
)
def _embed(xt_hbm, tablet_hbm, out_hbm, trow, xrows, orows, x_sp, tsem, xsems, wsems):
    sid = lax.axis_index("s")
    wid = sid * NC + lax.axis_index("c")

    for k in range((SEQ + NS - 1) // NS):
        t_load = k * NS + sid

        @pl.when(t_load < SEQ)
        def _load_x():
            pltpu.sync_copy(xt_hbm.at[t_load], x_sp.at[pl.ds(t_load * BN, BN)])

    plsc.subcore_barrier()

    wcp = [None, None]
    for dd in range(D_PER_W):
        d = wid * D_PER_W + dd
        tcp = pltpu.async_copy(tablet_hbm.at[d], trow, tsem)
        xcp = [None, None]
        xcp[0] = pltpu.async_copy(x_sp.at[pl.ds(0, BN)], xrows[0], xsems[0])
        tcp.wait()
        for t in range(SEQ):
            b = t % 2
            w = t % 2
            if t + 1 < SEQ:
                xcp[1 - b] = pltpu.async_copy(
                    x_sp.at[pl.ds((t + 1) * BN, BN)], xrows[1 - b], xsems[1 - b]
                )
            xcp[b].wait()
            if wcp[w] is not None:
                wcp[w].wait()
            xrow = xrows[b]
            orow = orows[w]

            @plsc.parallel_loop(0, BN, 16, unroll=8)
            def _gather(j):
                idx = xrow[pl.ds(j, 16)]
                orow[pl.ds(j, 16)] = plsc.load_gather(trow, [idx])

            wcp[w] = pltpu.async_copy(orow, out_hbm.at[t, d], wsems[w])
    wcp[0].wait()
    wcp[1].wait()


def kernel(x, table):
    out_t = _embed(x.T, table.T)
    return jnp.transpose(out_t, (2, 0, 1))
